# Initial kernel scaffold; baseline (speedup 1.0000x reference)
import jax
import jax.numpy as jnp
from jax.experimental import pallas as pl


def kernel(x, table):
    raise NotImplementedError("write your pallas kernel here")


# speedup vs baseline: 1.1127x; 1.1127x over previous
import functools

import jax
import jax.numpy as jnp
from jax import lax
from jax.experimental import pallas as pl
from jax.experimental.pallas import tpu as pltpu
from jax.experimental.pallas import tpu_sc as plsc

CHUNK = 1024


def _gather_call(n, D, num_cores, num_subcores):
    NW = num_cores * num_subcores
    per_w = n // NW
    n_ch = per_w // CHUNK
    assert n_ch % 2 == 0
    n_half = n_ch // 2
    mesh = plsc.VectorSubcoreMesh(core_axis_name="c", subcore_axis_name="s")

    @functools.partial(
        pl.kernel,
        mesh=mesh,
        out_type=jax.ShapeDtypeStruct((n, D), jnp.float32),
        scratch_types=[
            pltpu.VMEM((per_w,), jnp.int32),
            pltpu.VMEM((CHUNK, D), jnp.float32),
            pltpu.VMEM((CHUNK, D), jnp.float32),
            pltpu.SemaphoreType.DMA,
            pltpu.SemaphoreType.DMA,
            pltpu.SemaphoreType.DMA,
            pltpu.SemaphoreType.DMA,
        ],
        compiler_params=pltpu.CompilerParams(use_tc_tiling_on_sc=False),
    )
    def k(idx_hbm, tab_hbm, out_hbm, idx_v, rows0, rows1, g0, g1, s0, s1):
        wid = lax.axis_index("s") * num_cores + lax.axis_index("c")
        base = pl.multiple_of(wid * per_w, CHUNK)
        rows = (rows0, rows1)
        gsem = (g0, g1)
        ssem = (s0, s1)

        pltpu.sync_copy(idx_hbm.at[pl.ds(base, per_w)], idx_v)

        def start_gather(g, b):
            off = pl.multiple_of(g * CHUNK, CHUNK)
            pltpu.async_copy(
                tab_hbm.at[idx_v.at[pl.ds(off, CHUNK)]], rows[b], gsem[b]
            )

        def wait_gather(b):
            pltpu.make_async_copy(
                tab_hbm.at[idx_v.at[pl.ds(0, CHUNK)]], rows[b], gsem[b]
            ).wait()

        def start_scatter(g, b):
            off = pl.multiple_of(base + g * CHUNK, CHUNK)
            pltpu.async_copy(rows[b], out_hbm.at[pl.ds(off, CHUNK)], ssem[b])

        def wait_scatter(b):
            pltpu.make_async_copy(
                rows[b], out_hbm.at[pl.ds(0, CHUNK)], ssem[b]
            ).wait()

        start_gather(0, 0)
        start_gather(1, 1)

        def body(h, carry):
            gbase = 2 * h
            for b in range(2):
                g = gbase + b
                wait_gather(b)
                start_scatter(g, b)
                wait_scatter(b)
                start_gather(g + 2, b)
            return carry

        lax.fori_loop(0, n_half - 1, body, 0)

        for b in range(2):
            g = n_ch - 2 + b
            wait_gather(b)
            start_scatter(g, b)
            wait_scatter(b)

    return k


def kernel(x, table):
    B, F = x.shape
    V, D = table.shape
    n = B * F
    xf = x.reshape(n).astype(jnp.int32)
    info = plsc.get_sparse_core_info()
    out = _gather_call(n, D, info.num_cores, info.num_subcores)(xf, table)
    return out.reshape(B, F, D)


# speedup vs baseline: 4.6043x; 4.1381x over previous
import functools

import jax
import jax.numpy as jnp
from jax import lax
from jax.experimental import pallas as pl
from jax.experimental.pallas import tpu as pltpu
from jax.experimental.pallas import tpu_sc as plsc

B = 16384
F = 100
D = 32
BPW = 512
TCW = BPW // 128
TBLK = TCW * 8 * 128
FPLANE = (D // 8) * (B // 128) * 8 * 128


def _build_call(num_cores, num_subcores):
    mesh = plsc.VectorSubcoreMesh(core_axis_name="c", subcore_axis_name="s")

    @functools.partial(
        pl.kernel,
        mesh=mesh,
        out_type=jax.ShapeDtypeStruct((F * FPLANE,), jnp.float32),
        scratch_types=[
            pltpu.VMEM((BPW,), jnp.int32),
            pltpu.VMEM((BPW,), jnp.int32),
            pltpu.VMEM((BPW, D), jnp.float32),
            pltpu.VMEM((BPW, D), jnp.float32),
            pltpu.VMEM(((D // 8) * TBLK,), jnp.float32),
            pltpu.VMEM(((D // 8) * TBLK,), jnp.float32),
            pltpu.SemaphoreType.DMA,
            pltpu.SemaphoreType.DMA,
            pltpu.SemaphoreType.DMA,
            pltpu.SemaphoreType.DMA,
            pltpu.SemaphoreType.DMA,
            pltpu.SemaphoreType.DMA,
        ],
        compiler_params=pltpu.CompilerParams(
            use_tc_tiling_on_sc=False, needs_layout_passes=False
        ),
    )
    def k(xt_hbm, tab_hbm, out_hbm, i0, i1, g0, g1, t0, t1, is0, is1, gs0, gs1, ws0, ws1):
        wid = lax.axis_index("s") * num_cores + lax.axis_index("c")
        b0 = pl.multiple_of(wid * BPW, BPW)
        tc0 = wid * TCW
        I = (i0, i1)
        G = (g0, g1)
        T = (t0, t1)
        isem = (is0, is1)
        gsem = (gs0, gs1)
        wsem = (ws0, ws1)

        iota16 = lax.iota(jnp.int32, 16)
        base_lo = (iota16 // 8) * TBLK + (iota16 % 8) * 128
        base_hi = ((iota16 + 16) // 8) * TBLK + ((iota16 + 16) % 8) * 128

        def start_idx(f, p):
            pltpu.async_copy(xt_hbm.at[f, pl.ds(b0, BPW)], I[p], isem[p])

        def wait_idx(p):
            pltpu.make_async_copy(
                xt_hbm.at[0, pl.ds(b0, BPW)], I[p], isem[p]
            ).wait()

        def start_gather(p):
            pltpu.async_copy(tab_hbm.at[I[p]], G[p], gsem[p])

        def wait_gather(p):
            pltpu.make_async_copy(tab_hbm.at[I[p]], G[p], gsem[p]).wait()

        def start_write(f, p):
            for tr in range(D // 8):
                off = f * FPLANE + tr * (B // 128) * 1024 + tc0 * 1024
                pltpu.async_copy(
                    T[p].at[pl.ds(tr * TBLK, TBLK)],
                    out_hbm.at[pl.ds(off, TBLK)],
                    wsem[p],
                )

        def wait_write(p):
            for tr in range(D // 8):
                pltpu.make_async_copy(
                    T[p].at[pl.ds(tr * TBLK, TBLK)],
                    out_hbm.at[pl.ds(0, TBLK)],
                    wsem[p],
                ).wait()

        def transpose(p):
            g, t = G[p], T[p]
            for tcp in range(TCW):
                vlo0 = base_lo + tcp * 1024
                vhi0 = base_hi + tcp * 1024

                def body(c, carry, tcp=tcp):
                    vlo, vhi = carry
                    r = tcp * 128 + c
                    lo = g[r, pl.ds(0, 16)]
                    hi = g[r, pl.ds(16, 16)]
                    plsc.store_scatter(t, [vlo], lo)
                    plsc.store_scatter(t, [vhi], hi)
                    return (vlo + 1, vhi + 1)

                lax.fori_loop(0, 128, body, (vlo0, vhi0), unroll=16)

        def step(f, p, *, do_next_gather, do_idx_prefetch, do_wait_write):
            if do_next_gather:
                wait_idx(1 - p)
                start_gather(1 - p)
            wait_gather(p)
            if do_idx_prefetch:
                start_idx(f + 2, p)
            if do_wait_write:
                wait_write(p)
            transpose(p)
            start_write(f, p)

        pltpu.sync_copy(xt_hbm.at[0, pl.ds(b0, BPW)], I[0])
        start_idx(1, 1)
        start_gather(0)

        step(0, 0, do_next_gather=True, do_idx_prefetch=True, do_wait_write=False)
        step(1, 1, do_next_gather=True, do_idx_prefetch=True, do_wait_write=False)

        def body(h, carry):
            f = h * 2
            step(f, 0, do_next_gather=True, do_idx_prefetch=True, do_wait_write=True)
            step(f + 1, 1, do_next_gather=True, do_idx_prefetch=True, do_wait_write=True)
            return carry

        lax.fori_loop(1, F // 2 - 1, body, 0)

        step(F - 2, 0, do_next_gather=True, do_idx_prefetch=False, do_wait_write=True)
        step(F - 1, 1, do_next_gather=False, do_idx_prefetch=False, do_wait_write=True)

        wait_write(0)
        wait_write(1)

    return k


def kernel(x, table):
    xt = jnp.transpose(x, (1, 0)).astype(jnp.int32)
    info = plsc.get_sparse_core_info()
    out1 = _build_call(info.num_cores, info.num_subcores)(xt, table)
    out5 = jnp.reshape(out1, (F, D // 8, B // 128, 8, 128))
    return jnp.reshape(jnp.transpose(out5, (2, 4, 0, 1, 3)), (B, F, D))


# speedup vs baseline: 5.1671x; 1.1222x over previous
import functools

import jax
import jax.numpy as jnp
from jax import lax
from jax.experimental import pallas as pl
from jax.experimental.pallas import tpu as pltpu
from jax.experimental.pallas import tpu_sc as plsc

B = 16384
F = 100
D = 32
BPW = 512
TCW = BPW // 128
TBLK = TCW * 8 * 128
FPLANE = (D // 8) * (B // 128) * 8 * 128


def _build_call(num_cores, num_subcores):
    mesh = plsc.VectorSubcoreMesh(core_axis_name="c", subcore_axis_name="s")

    @functools.partial(
        pl.kernel,
        mesh=mesh,
        out_type=jax.ShapeDtypeStruct((F * FPLANE,), jnp.float32),
        scratch_types=[
            pltpu.VMEM((BPW,), jnp.int32),
            pltpu.VMEM((BPW,), jnp.int32),
            pltpu.VMEM((BPW, D), jnp.float32),
            pltpu.VMEM((BPW, D), jnp.float32),
            pltpu.VMEM(((D // 8) * TBLK,), jnp.float32),
            pltpu.VMEM(((D // 8) * TBLK,), jnp.float32),
            pltpu.SemaphoreType.DMA,
            pltpu.SemaphoreType.DMA,
            pltpu.SemaphoreType.DMA,
            pltpu.SemaphoreType.DMA,
            pltpu.SemaphoreType.DMA,
            pltpu.SemaphoreType.DMA,
        ],
        compiler_params=pltpu.CompilerParams(
            use_tc_tiling_on_sc=False, needs_layout_passes=False
        ),
    )
    def k(xt_hbm, tab_hbm, out_hbm, i0, i1, g0, g1, t0, t1, is0, is1, gs0, gs1, ws0, ws1):
        wid = lax.axis_index("s") * num_cores + lax.axis_index("c")
        b0 = pl.multiple_of(wid * BPW, BPW)
        tc0 = wid * TCW
        I = (i0, i1)
        G = (g0, g1)
        T = (t0, t1)
        isem = (is0, is1)
        gsem = (gs0, gs1)
        wsem = (ws0, ws1)

        iota16 = lax.iota(jnp.int32, 16)
        base_lo = (iota16 // 8) * TBLK + (iota16 % 8) * 128
        base_hi = ((iota16 + 16) // 8) * TBLK + ((iota16 + 16) % 8) * 128

        def start_idx(f, p):
            pltpu.async_copy(xt_hbm.at[f, pl.ds(b0, BPW)], I[p], isem[p])

        def wait_idx(p):
            pltpu.make_async_copy(
                xt_hbm.at[0, pl.ds(b0, BPW)], I[p], isem[p]
            ).wait()

        def start_gather(p):
            pltpu.async_copy(tab_hbm.at[I[p]], G[p], gsem[p])

        def wait_gather(p):
            pltpu.make_async_copy(tab_hbm.at[I[p]], G[p], gsem[p]).wait()

        def start_write(f, p):
            for tr in range(D // 8):
                off = f * FPLANE + tr * (B // 128) * 1024 + tc0 * 1024
                pltpu.async_copy(
                    T[p].at[pl.ds(tr * TBLK, TBLK)],
                    out_hbm.at[pl.ds(off, TBLK)],
                    wsem[p],
                )

        def wait_write(p):
            for tr in range(D // 8):
                pltpu.make_async_copy(
                    T[p].at[pl.ds(tr * TBLK, TBLK)],
                    out_hbm.at[pl.ds(0, TBLK)],
                    wsem[p],
                ).wait()

        def transpose(p):
            g, t = G[p], T[p]
            for tcp in range(TCW):
                vlo0 = base_lo + tcp * 1024
                vhi0 = base_hi + tcp * 1024

                @plsc.parallel_loop(0, 128, unroll=16)
                def body(c, tcp=tcp, vlo0=vlo0, vhi0=vhi0):
                    r = tcp * 128 + c
                    lo = g[r, pl.ds(0, 16)]
                    hi = g[r, pl.ds(16, 16)]
                    plsc.store_scatter(t, [vlo0 + c], lo)
                    plsc.store_scatter(t, [vhi0 + c], hi)

        def step(f, p, *, do_next_gather, do_idx_prefetch, do_wait_write):
            if do_next_gather:
                wait_idx(1 - p)
                start_gather(1 - p)
            wait_gather(p)
            if do_idx_prefetch:
                start_idx(f + 2, p)
            if do_wait_write:
                wait_write(p)
            transpose(p)
            start_write(f, p)

        pltpu.sync_copy(xt_hbm.at[0, pl.ds(b0, BPW)], I[0])
        start_idx(1, 1)
        start_gather(0)

        step(0, 0, do_next_gather=True, do_idx_prefetch=True, do_wait_write=False)
        step(1, 1, do_next_gather=True, do_idx_prefetch=True, do_wait_write=False)

        def body(h, carry):
            f = h * 2
            step(f, 0, do_next_gather=True, do_idx_prefetch=True, do_wait_write=True)
            step(f + 1, 1, do_next_gather=True, do_idx_prefetch=True, do_wait_write=True)
            return carry

        lax.fori_loop(1, F // 2 - 1, body, 0)

        step(F - 2, 0, do_next_gather=True, do_idx_prefetch=False, do_wait_write=True)
        step(F - 1, 1, do_next_gather=False, do_idx_prefetch=False, do_wait_write=True)

        wait_write(0)
        wait_write(1)

    return k


def kernel(x, table):
    xt = jnp.transpose(x, (1, 0)).astype(jnp.int32)
    info = plsc.get_sparse_core_info()
    out1 = _build_call(info.num_cores, info.num_subcores)(xt, table)
    out5 = jnp.reshape(out1, (F, D // 8, B // 128, 8, 128))
    return jnp.reshape(jnp.transpose(out5, (2, 4, 0, 1, 3)), (B, F, D))


# speedup vs baseline: 10.1907x; 1.9722x over previous
import functools

import jax
import jax.numpy as jnp
from jax import lax
from jax.experimental import pallas as pl
from jax.experimental.pallas import tpu as pltpu
from jax.experimental.pallas import tpu_sc as plsc

B = 16384
F = 100
D = 32
BPW = 512
TCW = BPW // 128
TBLK = TCW * 8 * 128
FPLANE = (D // 8) * (B // 128) * 8 * 128


def _build_call(num_cores, num_subcores):
    mesh = plsc.VectorSubcoreMesh(core_axis_name="c", subcore_axis_name="s")

    @functools.partial(
        pl.kernel,
        mesh=mesh,
        out_type=jax.ShapeDtypeStruct((F * FPLANE,), jnp.float32),
        scratch_types=[
            pltpu.VMEM((BPW,), jnp.int32),
            pltpu.VMEM((BPW,), jnp.int32),
            pltpu.VMEM((BPW, D), jnp.float32),
            pltpu.VMEM((BPW, D), jnp.float32),
            pltpu.VMEM(((D // 8) * TBLK,), jnp.float32),
            pltpu.VMEM(((D // 8) * TBLK,), jnp.float32),
            pltpu.SemaphoreType.DMA,
            pltpu.SemaphoreType.DMA,
            pltpu.SemaphoreType.DMA,
            pltpu.SemaphoreType.DMA,
            pltpu.SemaphoreType.DMA,
            pltpu.SemaphoreType.DMA,
        ],
        compiler_params=pltpu.CompilerParams(
            use_tc_tiling_on_sc=False, needs_layout_passes=False
        ),
    )
    def k(xt_hbm, tab_hbm, out_hbm, i0, i1, g0, g1, t0, t1, is0, is1, gs0, gs1, ws0, ws1):
        wid = lax.axis_index("s") * num_cores + lax.axis_index("c")
        b0 = pl.multiple_of(wid * BPW, BPW)
        tc0 = wid * TCW
        I = (i0, i1)
        G = (g0, g1)
        T = (t0, t1)
        isem = (is0, is1)
        gsem = (gs0, gs1)
        wsem = (ws0, ws1)

        iota16 = lax.iota(jnp.int32, 16)

        def start_idx(f, p):
            pltpu.async_copy(xt_hbm.at[f, pl.ds(b0, BPW)], I[p], isem[p])

        def wait_idx(p):
            pltpu.make_async_copy(
                xt_hbm.at[0, pl.ds(b0, BPW)], I[p], isem[p]
            ).wait()

        def start_gather(p):
            pltpu.async_copy(tab_hbm.at[I[p]], G[p], gsem[p])

        def wait_gather(p):
            pltpu.make_async_copy(tab_hbm.at[I[p]], G[p], gsem[p]).wait()

        def start_write(f, p):
            for tr in range(D // 8):
                off = f * FPLANE + tr * (B // 128) * 1024 + tc0 * 1024
                pltpu.async_copy(
                    T[p].at[pl.ds(tr * TBLK, TBLK)],
                    out_hbm.at[pl.ds(off, TBLK)],
                    wsem[p],
                )

        def wait_write(p):
            for tr in range(D // 8):
                pltpu.make_async_copy(
                    T[p].at[pl.ds(tr * TBLK, TBLK)],
                    out_hbm.at[pl.ds(0, TBLK)],
                    wsem[p],
                ).wait()

        def transpose(p):
            g, t = G[p], T[p]

            @plsc.parallel_loop(0, 512, unroll=4)
            def body(i):
                s = i & 15
                r0 = i >> 4 << 4
                dp = (iota16 + s) & 15
                tb = (dp >> 3) * TBLK + (dp & 7) * 128 + iota16
                rows = r0 + iota16
                off = (r0 >> 7) * 1024 + (r0 & 127)
                lo = plsc.load_gather(g, [rows, dp])
                hi = plsc.load_gather(g, [rows, dp + 16])
                plsc.store_scatter(t, [tb + off], lo)
                plsc.store_scatter(t, [tb + (off + 2 * TBLK)], hi)

        def step(f, p, *, do_next_gather, do_idx_prefetch, do_wait_write):
            if do_next_gather:
                wait_idx(1 - p)
                start_gather(1 - p)
            wait_gather(p)
            if do_idx_prefetch:
                start_idx(f + 2, p)
            if do_wait_write:
                wait_write(p)
            transpose(p)
            start_write(f, p)

        pltpu.sync_copy(xt_hbm.at[0, pl.ds(b0, BPW)], I[0])
        start_idx(1, 1)
        start_gather(0)

        step(0, 0, do_next_gather=True, do_idx_prefetch=True, do_wait_write=False)
        step(1, 1, do_next_gather=True, do_idx_prefetch=True, do_wait_write=False)

        def body(h, carry):
            f = h * 2
            step(f, 0, do_next_gather=True, do_idx_prefetch=True, do_wait_write=True)
            step(f + 1, 1, do_next_gather=True, do_idx_prefetch=True, do_wait_write=True)
            return carry

        lax.fori_loop(1, F // 2 - 1, body, 0)

        step(F - 2, 0, do_next_gather=True, do_idx_prefetch=False, do_wait_write=True)
        step(F - 1, 1, do_next_gather=False, do_idx_prefetch=False, do_wait_write=True)

        wait_write(0)
        wait_write(1)

    return k


def kernel(x, table):
    xt = jnp.transpose(x, (1, 0)).astype(jnp.int32)
    info = plsc.get_sparse_core_info()
    out1 = _build_call(info.num_cores, info.num_subcores)(xt, table)
    out5 = jnp.reshape(out1, (F, D // 8, B // 128, 8, 128))
    return jnp.reshape(jnp.transpose(out5, (2, 4, 0, 1, 3)), (B, F, D))
